# baseline (speedup 1.0000x reference)
import dataclasses
import functools

import jax
import jax.numpy as jnp
from jax import lax
from jax.experimental import pallas as pl
from jax.experimental.pallas import tpu as pltpu
from jax.experimental.pallas import tpu_sc as plsc

_NC = 2
_NS = 16
_NW = _NC * _NS
_L = 16


def _tree_sum(vals):
    while len(vals) > 1:
        nxt = [a + b for a, b in zip(vals[::2], vals[1::2])]
        if len(vals) % 2:
            nxt.append(vals[-1])
        vals = nxt
    return vals[0]


@functools.partial(jax.jit, static_argnames=("total_chunks", "c_nodes", "s"))
def _sc_mean_gather(idx, table_p, *, total_chunks, c_nodes, s):
    rows = c_nodes * s
    _, dw = table_p.shape
    b_out = total_chunks * c_nodes
    nbuf = 2
    t_max = -(-total_chunks // _NW)
    t_max += (-t_max) % nbuf
    scale = jnp.float32(1.0 / (float(s) + 1e-15))

    mesh = plsc.VectorSubcoreMesh(core_axis_name="c", subcore_axis_name="s",
                                  num_cores=_NC, num_subcores=_NS)

    cp = pltpu.CompilerParams()
    if "needs_layout_passes" in pltpu.CompilerParams.__dataclass_fields__:
        cp = dataclasses.replace(cp, needs_layout_passes=False)
    cp = dataclasses.replace(cp, use_tc_tiling_on_sc=False)

    @functools.partial(
        pl.kernel,
        out_type=jax.ShapeDtypeStruct((b_out, 2 * dw), jnp.float32),
        mesh=mesh,
        compiler_params=cp,
        scratch_types=(
            [pltpu.VMEM((t_max * rows,), jnp.int32)]
            + [pltpu.VMEM((rows, dw), jnp.int32) for _ in range(nbuf)]
            + [pltpu.VMEM((c_nodes, 2 * dw), jnp.float32) for _ in range(nbuf)]
            + [pltpu.SemaphoreType.DMA for _ in range(2 * nbuf)]
        ),
    )
    def k(idx_hbm, table_hbm, out_hbm, idx_v, *bufs):
        rows_v = bufs[:nbuf]
        out_v = bufs[nbuf:2 * nbuf]
        gsem = bufs[2 * nbuf:3 * nbuf]
        osem = bufs[3 * nbuf:4 * nbuf]

        wid = lax.axis_index("c") * _NS + lax.axis_index("s")
        start_w = (wid * total_chunks) // _NW
        n_w = ((wid + 1) * total_chunks) // _NW - start_w
        nm1 = n_w - 1

        pltpu.sync_copy(idx_hbm.at[pl.ds(start_w * rows, t_max * rows)],
                        idx_v)

        def gather(step_lc, b):
            return pltpu.make_async_copy(
                table_hbm.at[idx_v.at[pl.ds(step_lc * rows, rows)]],
                rows_v[b], gsem[b])

        def out_store(step_lc, b):
            return pltpu.make_async_copy(
                out_v[b],
                out_hbm.at[pl.ds((start_w + step_lc) * c_nodes, c_nodes)],
                osem[b])

        for b in range(nbuf):
            gather(lax.min(jnp.int32(b), nm1), b).start()

        @pl.loop(0, t_max // nbuf)
        def _steps(t):
            for b in range(nbuf):
                i = nbuf * t + b
                lc = lax.min(i, nm1)
                gather(lc, b).wait()

                @pl.when(t >= 1)
                def _():
                    out_store(lax.min(i - nbuf, nm1), b).wait()

                rv, ov = rows_v[b], out_v[b]
                for n in range(c_nodes):
                    for c in range(dw // _L):
                        sl = pl.ds(c * _L, _L)
                        terms = [plsc.bitcast(rv[n * s + kk, sl], jnp.bfloat16)
                                 for kk in range(s)]
                        lo_f, hi_f = plsc.unpack(
                            _tree_sum(terms), format=plsc.PackFormat.INTERLEAVED)
                        ov[n, sl] = lo_f * scale
                        ov[n, pl.ds(dw + c * _L, _L)] = hi_f * scale

                out_store(lc, b).start()
                gather(lax.min(i + nbuf, nm1), b).start()

        for b in range(nbuf):
            gather(nm1, b).wait()
            out_store(nm1, b).wait()

    return k(idx, table_p)


def kernel(nodes, to_neighs, table):
    b, s = to_neighs.shape
    v, d = table.shape
    c_nodes = 8
    total_chunks = -(-b // c_nodes)
    idx = to_neighs.reshape(-1)
    if total_chunks * c_nodes != b:
        idx = jnp.pad(idx, (0, (total_chunks * c_nodes - b) * s))
    t_max = -(-total_chunks // _NW)
    t_max += (-t_max) % 2
    needed = (((_NW - 1) * total_chunks) // _NW + t_max) * c_nodes * s
    if needed > idx.shape[0]:
        idx = jnp.pad(idx, (0, needed - idx.shape[0]))
    u = lax.bitcast_convert_type(table, jnp.int32)

    def _rnd(x):
        return (x + jnp.int32(0x7FFF) + ((x >> 16) & 1)) >> 16

    table_p = ((_rnd(u[:, :d // 2]) & jnp.int32(0xFFFF))
               | (_rnd(u[:, d // 2:]) << 16))
    out = _sc_mean_gather(idx, table_p, total_chunks=total_chunks,
                          c_nodes=c_nodes, s=s)
    return out[:b] if total_chunks * c_nodes != b else out


# speedup vs baseline: 1.0179x; 1.0179x over previous
import dataclasses
import functools

import jax
import jax.numpy as jnp
from jax import lax
from jax.experimental import pallas as pl
from jax.experimental.pallas import tpu as pltpu
from jax.experimental.pallas import tpu_sc as plsc

_NC = 2
_NS = 16
_NW = _NC * _NS
_L = 16


def _tree_sum(vals):
    while len(vals) > 1:
        nxt = [a + b for a, b in zip(vals[::2], vals[1::2])]
        if len(vals) % 2:
            nxt.append(vals[-1])
        vals = nxt
    return vals[0]


@functools.partial(jax.jit, static_argnames=("total_chunks", "c_nodes", "s"))
def _sc_mean_gather(idx, table_p, *, total_chunks, c_nodes, s):
    rows = c_nodes * s
    _, d = table_p.shape
    b_out = total_chunks * c_nodes
    nbuf = 2
    t_max = -(-total_chunks // _NW)
    t_max += (-t_max) % nbuf
    scale = jnp.float32(1.0 / (float(s) + 1e-15))

    mesh = plsc.VectorSubcoreMesh(core_axis_name="c", subcore_axis_name="s",
                                  num_cores=_NC, num_subcores=_NS)

    cp = pltpu.CompilerParams()
    if "needs_layout_passes" in pltpu.CompilerParams.__dataclass_fields__:
        cp = dataclasses.replace(cp, needs_layout_passes=False)
    cp = dataclasses.replace(cp, use_tc_tiling_on_sc=False)

    @functools.partial(
        pl.kernel,
        out_type=jax.ShapeDtypeStruct((b_out, d), jnp.float32),
        mesh=mesh,
        compiler_params=cp,
        scratch_types=(
            [pltpu.VMEM((t_max * rows,), jnp.int32)]
            + [pltpu.VMEM((rows, d), jnp.bfloat16) for _ in range(nbuf)]
            + [pltpu.VMEM((c_nodes, d), jnp.float32) for _ in range(nbuf)]
            + [pltpu.SemaphoreType.DMA for _ in range(2 * nbuf)]
        ),
    )
    def k(idx_hbm, table_hbm, out_hbm, idx_v, *bufs):
        rows_v = bufs[:nbuf]
        out_v = bufs[nbuf:2 * nbuf]
        gsem = bufs[2 * nbuf:3 * nbuf]
        osem = bufs[3 * nbuf:4 * nbuf]

        iota2 = lax.iota(jnp.int32, _L) * 2
        wid = lax.axis_index("c") * _NS + lax.axis_index("s")
        start_w = (wid * total_chunks) // _NW
        n_w = ((wid + 1) * total_chunks) // _NW - start_w
        nm1 = n_w - 1

        pltpu.sync_copy(idx_hbm.at[pl.ds(start_w * rows, t_max * rows)],
                        idx_v)

        def gather(step_lc, b):
            return pltpu.make_async_copy(
                table_hbm.at[idx_v.at[pl.ds(step_lc * rows, rows)]],
                rows_v[b], gsem[b])

        def out_store(step_lc, b):
            return pltpu.make_async_copy(
                out_v[b],
                out_hbm.at[pl.ds((start_w + step_lc) * c_nodes, c_nodes)],
                osem[b])

        for b in range(nbuf):
            gather(lax.min(jnp.int32(b), nm1), b).start()

        @pl.loop(0, t_max // nbuf)
        def _steps(t):
            for b in range(nbuf):
                i = nbuf * t + b
                lc = lax.min(i, nm1)
                gather(lc, b).wait()

                @pl.when(t >= 1)
                def _():
                    out_store(lax.min(i - nbuf, nm1), b).wait()

                rv, ov = rows_v[b], out_v[b]
                for n in range(c_nodes):
                    for c in range(d // (2 * _L)):
                        sl = pl.ds(c * 2 * _L, 2 * _L)
                        terms = [rv[n * s + kk, sl] for kk in range(s)]
                        ev_f, od_f = plsc.unpack(
                            _tree_sum(terms), format=plsc.PackFormat.INTERLEAVED)
                        plsc.store_scatter(ov.at[n], [iota2 + (c * 2 * _L)],
                                           ev_f * scale)
                        plsc.store_scatter(ov.at[n], [iota2 + (c * 2 * _L + 1)],
                                           od_f * scale)

                out_store(lc, b).start()
                gather(lax.min(i + nbuf, nm1), b).start()

        for b in range(nbuf):
            gather(nm1, b).wait()
            out_store(nm1, b).wait()

    return k(idx, table_p)


def kernel(nodes, to_neighs, table):
    b, s = to_neighs.shape
    v, d = table.shape
    c_nodes = 8
    total_chunks = -(-b // c_nodes)
    idx = to_neighs.reshape(-1)
    if total_chunks * c_nodes != b:
        idx = jnp.pad(idx, (0, (total_chunks * c_nodes - b) * s))
    t_max = -(-total_chunks // _NW)
    t_max += (-t_max) % 2
    needed = (((_NW - 1) * total_chunks) // _NW + t_max) * c_nodes * s
    if needed > idx.shape[0]:
        idx = jnp.pad(idx, (0, needed - idx.shape[0]))
    table_p = table.astype(jnp.bfloat16)
    out = _sc_mean_gather(idx, table_p, total_chunks=total_chunks,
                          c_nodes=c_nodes, s=s)
    return out[:b] if total_chunks * c_nodes != b else out
